# baseline (speedup 1.0000x reference)
import functools

import jax
import jax.numpy as jnp
from jax.experimental import pallas as pl
from jax.experimental.pallas import tpu as pltpu

_R = 48


def _sum_body(x_ref, out_ref):
    si = pl.program_id(1)
    s = jnp.sum(x_ref[0], axis=0, keepdims=True)
    acc = jnp.broadcast_to(s, out_ref.shape[1:])

    @pl.when(si == 0)
    def _init():
        out_ref[0] = acc

    @pl.when(si != 0)
    def _acc():
        out_ref[0] += acc


def _mlp_topk_body(s_ref, w1_ref, b1_ref, w2_ref, b2_ref, p_ref, *, n_spatial):
    b, _, c = s_ref.shape
    r = _R
    y0 = s_ref[:, 0, :] * (1.0 / n_spatial)
    h = jax.lax.dot_general(y0, w1_ref[...], (((1,), (1,)), ((), ())),
                            preferred_element_type=jnp.float32) + b1_ref[...]
    h = jnp.where(h >= 0, h, 0.01 * h)
    z = jax.lax.dot_general(h, w2_ref[...], (((1,), (1,)), ((), ())),
                            preferred_element_type=jnp.float32) + b2_ref[...]
    s = jax.nn.sigmoid(z)
    si = s[:, :, None]
    sj = s[:, None, :]
    ii = jax.lax.broadcasted_iota(jnp.int32, (b, c, c), 1)
    jj = jax.lax.broadcasted_iota(jnp.int32, (b, c, c), 2)
    beats = (sj > si) | ((sj == si) & (jj < ii))
    rank = jnp.sum(beats.astype(jnp.int32), axis=2)
    pp = jax.lax.broadcasted_iota(jnp.int32, (b, c, r), 2)
    p_ref[...] = (rank[:, :, None] == pp).astype(jnp.float32)


def _gather_mm_body(x_ref, p_ref, o_ref):
    o_ref[0] = jax.lax.dot_general(
        x_ref[0], p_ref[0], (((1,), (0,)), ((), ())),
        precision=jax.lax.Precision.HIGHEST,
        preferred_element_type=jnp.float32)


def kernel(x, w1, b1, w2, b2):
    b, c, d, h, w = x.shape
    n = d * h * w
    xm = jnp.transpose(x, (0, 2, 3, 4, 1)).reshape(b, n, c)

    m_blk1 = 16384
    sums = pl.pallas_call(
        _sum_body,
        grid=(b, n // m_blk1),
        in_specs=[pl.BlockSpec((1, m_blk1, c), lambda bi, si: (bi, si, 0))],
        out_specs=pl.BlockSpec((1, 8, c), lambda bi, si: (bi, 0, 0)),
        out_shape=jax.ShapeDtypeStruct((b, 8, c), jnp.float32),
    )(xm)

    pmat = pl.pallas_call(
        functools.partial(_mlp_topk_body, n_spatial=n),
        out_shape=jax.ShapeDtypeStruct((b, c, _R), jnp.float32),
    )(sums, w1, b1.reshape(1, c), w2, b2.reshape(1, c))

    m_blk2 = 8192
    out_t = pl.pallas_call(
        _gather_mm_body,
        grid=(b, n // m_blk2),
        in_specs=[
            pl.BlockSpec((1, m_blk2, c), lambda bi, si: (bi, si, 0)),
            pl.BlockSpec((1, c, _R), lambda bi, si: (bi, 0, 0)),
        ],
        out_specs=pl.BlockSpec((1, m_blk2, _R), lambda bi, si: (bi, si, 0)),
        out_shape=jax.ShapeDtypeStruct((b, n, _R), jnp.float32),
    )(xm, pmat)

    return jnp.transpose(out_t.reshape(b, d, h, w, _R), (0, 4, 1, 2, 3))


# speedup vs baseline: 1.1560x; 1.1560x over previous
import functools

import jax
import jax.numpy as jnp
from jax.experimental import pallas as pl
from jax.experimental.pallas import tpu as pltpu

_R = 48


def _sum_body(x_ref, out_ref):
    si = pl.program_id(1)
    s = jnp.sum(x_ref[0], axis=0, keepdims=True)
    acc = jnp.broadcast_to(s, out_ref.shape[1:])

    @pl.when(si == 0)
    def _init():
        out_ref[0] = acc

    @pl.when(si != 0)
    def _acc():
        out_ref[0] += acc


def _mlp_topk_body(s_ref, w1_ref, b1_ref, w2_ref, b2_ref, p_ref, *, n_spatial):
    b, _, c = s_ref.shape
    r = _R
    y0 = s_ref[:, 0, :] * (1.0 / n_spatial)
    h = jax.lax.dot_general(y0, w1_ref[...], (((1,), (1,)), ((), ())),
                            preferred_element_type=jnp.float32) + b1_ref[...]
    h = jnp.where(h >= 0, h, 0.01 * h)
    z = jax.lax.dot_general(h, w2_ref[...], (((1,), (1,)), ((), ())),
                            preferred_element_type=jnp.float32) + b2_ref[...]
    s = jax.nn.sigmoid(z)
    si = s[:, :, None]
    sj = s[:, None, :]
    ii = jax.lax.broadcasted_iota(jnp.int32, (b, c, c), 1)
    jj = jax.lax.broadcasted_iota(jnp.int32, (b, c, c), 2)
    beats = (sj > si) | ((sj == si) & (jj < ii))
    rank = jnp.sum(beats.astype(jnp.int32), axis=2)
    pp = jax.lax.broadcasted_iota(jnp.int32, (b, c, r), 2)
    im = jax.lax.broadcasted_iota(jnp.int32, (b, c, r), 1)
    onehot = (rank[:, :, None] == pp).astype(jnp.int32)
    p_ref[...] = jnp.sum(onehot * im, axis=1)[:, None, :]


def _gather_mm_body(x_ref, p_ref, o_ref):
    a = x_ref[0]
    idx = jnp.broadcast_to(p_ref[0], (a.shape[0], _R))
    o_ref[0] = jnp.take_along_axis(a, idx, axis=1)


def kernel(x, w1, b1, w2, b2):
    b, c, d, h, w = x.shape
    n = d * h * w
    xm = jnp.transpose(x, (0, 2, 3, 4, 1)).reshape(b, n, c)

    m_blk1 = 16384
    sums = pl.pallas_call(
        _sum_body,
        grid=(b, n // m_blk1),
        in_specs=[pl.BlockSpec((1, m_blk1, c), lambda bi, si: (bi, si, 0))],
        out_specs=pl.BlockSpec((1, 8, c), lambda bi, si: (bi, 0, 0)),
        out_shape=jax.ShapeDtypeStruct((b, 8, c), jnp.float32),
    )(xm)

    idx3 = pl.pallas_call(
        functools.partial(_mlp_topk_body, n_spatial=n),
        out_shape=jax.ShapeDtypeStruct((b, 1, _R), jnp.int32),
    )(sums, w1, b1.reshape(1, c), w2, b2.reshape(1, c))

    m_blk2 = 8192
    out_t = pl.pallas_call(
        _gather_mm_body,
        grid=(b, n // m_blk2),
        in_specs=[
            pl.BlockSpec((1, m_blk2, c), lambda bi, si: (bi, si, 0)),
            pl.BlockSpec((1, 1, _R), lambda bi, si: (bi, 0, 0)),
        ],
        out_specs=pl.BlockSpec((1, m_blk2, _R), lambda bi, si: (bi, si, 0)),
        out_shape=jax.ShapeDtypeStruct((b, n, _R), jnp.float32),
    )(xm, idx3)

    return jnp.transpose(out_t.reshape(b, d, h, w, _R), (0, 4, 1, 2, 3))


# speedup vs baseline: 1.4426x; 1.2479x over previous
import functools

import jax
import jax.numpy as jnp
from jax.experimental import pallas as pl
from jax.experimental.pallas import tpu as pltpu

_R = 48


def _sum_body(x_ref, out_ref):
    si = pl.program_id(1)
    s = jnp.sum(x_ref[0], axis=0, keepdims=True)
    acc = jnp.broadcast_to(s, out_ref.shape[1:])

    @pl.when(si == 0)
    def _init():
        out_ref[0] = acc

    @pl.when(si != 0)
    def _acc():
        out_ref[0] += acc


def _mlp_topk_body(s_ref, w1_ref, b1_ref, w2_ref, b2_ref, p_ref, *, n_spatial):
    b, _, c = s_ref.shape
    r = _R
    y0 = s_ref[:, 0, :] * (1.0 / n_spatial)
    h = jax.lax.dot_general(y0, w1_ref[...], (((1,), (1,)), ((), ())),
                            preferred_element_type=jnp.float32) + b1_ref[...]
    h = jnp.where(h >= 0, h, 0.01 * h)
    z = jax.lax.dot_general(h, w2_ref[...], (((1,), (1,)), ((), ())),
                            preferred_element_type=jnp.float32) + b2_ref[...]
    s = jax.nn.sigmoid(z)
    si = s[:, :, None]
    sj = s[:, None, :]
    ii = jax.lax.broadcasted_iota(jnp.int32, (b, c, c), 1)
    jj = jax.lax.broadcasted_iota(jnp.int32, (b, c, c), 2)
    beats = (sj > si) | ((sj == si) & (jj < ii))
    rank = jnp.sum(beats.astype(jnp.int32), axis=2)
    pp = jax.lax.broadcasted_iota(jnp.int32, (b, c, r), 2)
    im = jax.lax.broadcasted_iota(jnp.int32, (b, c, r), 1)
    onehot = (rank[:, :, None] == pp).astype(jnp.int32)
    p_ref[...] = jnp.sum(onehot * im, axis=1)[:, None, :]


def _gather_tr_body(x_ref, p_ref, o_ref):
    hh, ww, _ = x_ref.shape[1:]
    a = x_ref[0]
    idx = jnp.broadcast_to(p_ref[0, 0][None, None, :], (hh, ww, _R))
    g = jnp.take_along_axis(a, idx, axis=2)
    o_ref[0, :, 0] = jnp.transpose(g, (2, 0, 1))


def kernel(x, w1, b1, w2, b2):
    b, c, d, h, w = x.shape
    n = d * h * w
    xm = jnp.transpose(x, (0, 2, 3, 4, 1)).reshape(b, n, c)

    m_blk1 = 16384
    sums = pl.pallas_call(
        _sum_body,
        grid=(b, n // m_blk1),
        in_specs=[pl.BlockSpec((1, m_blk1, c), lambda bi, si: (bi, si, 0))],
        out_specs=pl.BlockSpec((1, 8, c), lambda bi, si: (bi, 0, 0)),
        out_shape=jax.ShapeDtypeStruct((b, 8, c), jnp.float32),
    )(xm)

    idx3 = pl.pallas_call(
        functools.partial(_mlp_topk_body, n_spatial=n),
        out_shape=jax.ShapeDtypeStruct((b, 1, _R), jnp.int32),
    )(sums, w1, b1.reshape(1, c), w2, b2.reshape(1, c))

    xv = xm.reshape(b, d * h, w, c)
    out = pl.pallas_call(
        _gather_tr_body,
        grid=(b, d),
        in_specs=[
            pl.BlockSpec((1, h, w, c), lambda bi, di: (bi, di, 0, 0)),
            pl.BlockSpec((1, 1, _R), lambda bi, di: (bi, 0, 0)),
        ],
        out_specs=pl.BlockSpec((1, _R, 1, h, w), lambda bi, di: (bi, 0, di, 0, 0)),
        out_shape=jax.ShapeDtypeStruct((b, _R, d, h, w), jnp.float32),
    )(xv, idx3)
    return out


# speedup vs baseline: 1.6292x; 1.1294x over previous
import functools

import jax
import jax.numpy as jnp
from jax.experimental import pallas as pl
from jax.experimental.pallas import tpu as pltpu

_R = 48


def _sum_body(x_ref, out_ref):
    si = pl.program_id(1)
    s = jnp.sum(x_ref[0], axis=0, keepdims=True)
    acc = jnp.broadcast_to(s, out_ref.shape[1:])

    @pl.when(si == 0)
    def _init():
        out_ref[0] = acc

    @pl.when(si != 0)
    def _acc():
        out_ref[0] += acc


def _mlp_topk_body(s_ref, w1_ref, b1_ref, w2_ref, b2_ref, p_ref, *, n_spatial):
    b, _, c = s_ref.shape
    r = _R
    y0 = s_ref[:, 0, :] * (1.0 / n_spatial)
    h = jax.lax.dot_general(y0, w1_ref[...], (((1,), (1,)), ((), ())),
                            preferred_element_type=jnp.float32) + b1_ref[...]
    h = jnp.where(h >= 0, h, 0.01 * h)
    z = jax.lax.dot_general(h, w2_ref[...], (((1,), (1,)), ((), ())),
                            preferred_element_type=jnp.float32) + b2_ref[...]
    s = jax.nn.sigmoid(z)
    si = s[:, :, None]
    sj = s[:, None, :]
    ii = jax.lax.broadcasted_iota(jnp.int32, (b, c, c), 1)
    jj = jax.lax.broadcasted_iota(jnp.int32, (b, c, c), 2)
    beats = (sj > si) | ((sj == si) & (jj < ii))
    rank = jnp.sum(beats.astype(jnp.int32), axis=2)
    pp = jax.lax.broadcasted_iota(jnp.int32, (b, c, r), 2)
    im = jax.lax.broadcasted_iota(jnp.int32, (b, c, r), 1)
    onehot = (rank[:, :, None] == pp).astype(jnp.int32)
    p_ref[...] = jnp.sum(onehot * im, axis=1)[:, None, :]


def _gather_tr_body(x_ref, p_ref, o_ref):
    hd, ww, _ = x_ref.shape[1:]
    a = x_ref[0]
    idx = jnp.broadcast_to(p_ref[0, 0][None, None, :], (hd, ww, _R))
    g = jnp.take_along_axis(a, idx, axis=2)
    d_blk = o_ref.shape[2]
    gt = jnp.transpose(g, (2, 0, 1))
    o_ref[0] = gt.reshape(_R, d_blk, hd // d_blk, ww)


def kernel(x, w1, b1, w2, b2):
    b, c, d, h, w = x.shape
    n = d * h * w
    xm = jnp.transpose(x, (0, 2, 3, 4, 1)).reshape(b, n, c)

    m_blk1 = 16384
    sums = pl.pallas_call(
        _sum_body,
        grid=(b, n // m_blk1),
        in_specs=[pl.BlockSpec((1, m_blk1, c), lambda bi, si: (bi, si, 0))],
        out_specs=pl.BlockSpec((1, 8, c), lambda bi, si: (bi, 0, 0)),
        out_shape=jax.ShapeDtypeStruct((b, 8, c), jnp.float32),
    )(xm)

    idx3 = pl.pallas_call(
        functools.partial(_mlp_topk_body, n_spatial=n),
        out_shape=jax.ShapeDtypeStruct((b, 1, _R), jnp.int32),
    )(sums, w1, b1.reshape(1, c), w2, b2.reshape(1, c))

    d_blk = 2
    xv = xm.reshape(b, d * h, w, c)
    out = pl.pallas_call(
        _gather_tr_body,
        grid=(b, d // d_blk),
        in_specs=[
            pl.BlockSpec((1, d_blk * h, w, c), lambda bi, di: (bi, di, 0, 0)),
            pl.BlockSpec((1, 1, _R), lambda bi, di: (bi, 0, 0)),
        ],
        out_specs=pl.BlockSpec(
            (1, _R, d_blk, h, w), lambda bi, di: (bi, 0, di, 0, 0)),
        out_shape=jax.ShapeDtypeStruct((b, _R, d, h, w), jnp.float32),
    )(xv, idx3)
    return out


# speedup vs baseline: 1.7238x; 1.0580x over previous
import functools

import jax
import jax.numpy as jnp
from jax.experimental import pallas as pl
from jax.experimental.pallas import tpu as pltpu

_R = 48


def _sum_body(x_ref, out_ref):
    si = pl.program_id(1)
    s = jnp.sum(x_ref[0], axis=0, keepdims=True)
    acc = jnp.broadcast_to(s, out_ref.shape[1:])

    @pl.when(si == 0)
    def _init():
        out_ref[0] = acc

    @pl.when(si != 0)
    def _acc():
        out_ref[0] += acc


def _mlp_topk_body(s_ref, w1_ref, b1_ref, w2_ref, b2_ref, p_ref, *, n_spatial):
    b, _, c = s_ref.shape
    r = _R
    y0 = s_ref[:, 0, :] * (1.0 / n_spatial)
    h = jax.lax.dot_general(y0, w1_ref[...], (((1,), (1,)), ((), ())),
                            preferred_element_type=jnp.float32) + b1_ref[...]
    h = jnp.where(h >= 0, h, 0.01 * h)
    z = jax.lax.dot_general(h, w2_ref[...], (((1,), (1,)), ((), ())),
                            preferred_element_type=jnp.float32) + b2_ref[...]
    s = jax.nn.sigmoid(z)
    si = s[:, :, None]
    sj = s[:, None, :]
    ii = jax.lax.broadcasted_iota(jnp.int32, (b, c, c), 1)
    jj = jax.lax.broadcasted_iota(jnp.int32, (b, c, c), 2)
    beats = (sj > si) | ((sj == si) & (jj < ii))
    rank = jnp.sum(beats.astype(jnp.int32), axis=2)
    pp = jax.lax.broadcasted_iota(jnp.int32, (b, c, r), 2)
    im = jax.lax.broadcasted_iota(jnp.int32, (b, c, r), 1)
    onehot = (rank[:, :, None] == pp).astype(jnp.int32)
    p_ref[...] = jnp.sum(onehot * im, axis=1)[:, None, :]


def _gather_tr_body(x_ref, p_ref, o_ref):
    hd, ww, _ = x_ref.shape[1:]
    a = x_ref[0]
    idx = jnp.broadcast_to(p_ref[0, 0][None, None, :], (hd, ww, _R))
    g = jnp.take_along_axis(a, idx, axis=2)
    d_blk = o_ref.shape[2]
    gt = jnp.transpose(g, (2, 0, 1))
    o_ref[0] = gt.reshape(_R, d_blk, hd // d_blk, ww)


def kernel(x, w1, b1, w2, b2):
    b, c, d, h, w = x.shape
    n = d * h * w
    xm = jnp.transpose(x, (0, 2, 3, 4, 1)).reshape(b, n, c)

    m_blk1 = 16384
    sums = pl.pallas_call(
        _sum_body,
        grid=(b, n // m_blk1),
        in_specs=[pl.BlockSpec((1, m_blk1, c), lambda bi, si: (bi, si, 0))],
        out_specs=pl.BlockSpec((1, 8, c), lambda bi, si: (bi, 0, 0)),
        out_shape=jax.ShapeDtypeStruct((b, 8, c), jnp.float32),
    )(xm)

    idx3 = pl.pallas_call(
        functools.partial(_mlp_topk_body, n_spatial=n),
        out_shape=jax.ShapeDtypeStruct((b, 1, _R), jnp.int32),
    )(sums, w1, b1.reshape(1, c), w2, b2.reshape(1, c))

    d_blk = 4
    xv = xm.reshape(b, d * h, w, c)
    out = pl.pallas_call(
        _gather_tr_body,
        grid=(b, d // d_blk),
        in_specs=[
            pl.BlockSpec((1, d_blk * h, w, c), lambda bi, di: (bi, di, 0, 0)),
            pl.BlockSpec((1, 1, _R), lambda bi, di: (bi, 0, 0)),
        ],
        out_specs=pl.BlockSpec(
            (1, _R, d_blk, h, w), lambda bi, di: (bi, 0, di, 0, 0)),
        out_shape=jax.ShapeDtypeStruct((b, _R, d, h, w), jnp.float32),
    )(xv, idx3)
    return out
